# baseline (speedup 1.0000x reference)
import functools

import jax
import jax.numpy as jnp
from jax import lax
from jax.experimental import pallas as pl
from jax.experimental.pallas import tpu as pltpu
from jax.experimental.pallas import tpu_sc as plsc

TBW = 96
PW = 40
GW = 128
CH = 128
TE = 2000
TN = 2000


def _sc_mesh():
  return plsc.VectorSubcoreMesh(core_axis_name="c", subcore_axis_name="s")


_SC_PARAMS = pltpu.CompilerParams(use_tc_tiling_on_sc=False)


def _sc_gather2(table, idx_d, idx_s):
  e = idx_d.shape[1]
  out_t = jax.ShapeDtypeStruct((e, TBW), jnp.bfloat16)

  @functools.partial(
      pl.kernel,
      out_type=(out_t, out_t),
      mesh=_sc_mesh(),
      compiler_params=_SC_PARAMS,
  )
  def k(t_hbm, id_hbm, is_hbm, od_hbm, os_hbm):
    def body(id_v, is_v, od_v, os_v):
      pltpu.sync_copy(t_hbm.at[id_v.at[0]], od_v)
      pltpu.sync_copy(t_hbm.at[is_v.at[0]], os_v)

    pltpu.emit_pipeline(
        body,
        grid=(e // GW,),
        in_specs=[
            pl.BlockSpec((1, GW), lambda i: (0, i)),
            pl.BlockSpec((1, GW), lambda i: (0, i)),
        ],
        out_specs=[
            pl.BlockSpec((GW, TBW), lambda i: (i, 0)),
            pl.BlockSpec((GW, TBW), lambda i: (i, 0)),
        ],
        core_axis_name=("c", "s"),
        dimension_semantics=(pltpu.PARALLEL,),
    )(id_hbm, is_hbm, od_hbm, os_hbm)

  return k(table, idx_d, idx_s)


def _sc_scatter_add(payload, idx2d, zeros_n):
  e = payload.shape[1]
  n = zeros_n.shape[0]
  total_ch = e // CH
  rows_per = n // 16

  @functools.partial(
      pl.kernel,
      out_type=jax.ShapeDtypeStruct((2, n, PW), jnp.float32),
      mesh=_sc_mesh(),
      compiler_params=_SC_PARAMS,
      scratch_types=[
          pltpu.VMEM_SHARED((n, PW), jnp.float32),
          pltpu.VMEM((CH, PW), jnp.float32),
          pltpu.VMEM((1, CH), jnp.int32),
          pltpu.SemaphoreType.DMA,
      ],
  )
  def k(p_hbm, i_hbm, z_hbm, o_hbm, acc_sh, pay, idx, ss):
    c = lax.axis_index("c")
    s = lax.axis_index("s")
    pltpu.sync_copy(
        z_hbm.at[pl.ds(s * rows_per, rows_per)],
        acc_sh.at[pl.ds(s * rows_per, rows_per)],
    )
    plsc.subcore_barrier()

    n_trips = (total_ch - s + 15) // 16

    def trip(t, carry):
      ch = s + 16 * t
      pltpu.sync_copy(i_hbm.at[pl.ds(ch, 1)], idx)
      pltpu.sync_copy(p_hbm.at[c].at[pl.ds(ch * CH, CH)], pay)
      pltpu.async_copy(pay, acc_sh.at[idx.at[0]], ss, add=True).wait()
      return carry

    lax.fori_loop(0, n_trips, trip, 0)

    plsc.subcore_barrier()
    pltpu.sync_copy(
        acc_sh.at[pl.ds(s * rows_per, rows_per)],
        o_hbm.at[c].at[pl.ds(s * rows_per, rows_per)],
    )

  return k(payload, idx2d, zeros_n)


def _pack_table(h, pos):
  ph = pos.astype(jnp.bfloat16).astype(jnp.float32)
  plo = pos - ph
  b = h.shape[0]
  t32 = jnp.concatenate(
      [h, ph, plo, jnp.zeros((b, TBW - 70), jnp.float32)], axis=1)
  return t32.astype(jnp.bfloat16)


def _tc_lin_in(x, pos, w, b):
  n, in_dim = x.shape

  def body(x_ref, p_ref, w_ref, b_ref, h_ref, t_ref):
    h = jnp.dot(x_ref[...], w_ref[...], preferred_element_type=jnp.float32)
    h = h + b_ref[...]
    h_ref[...] = h
    t_ref[...] = _pack_table(h, p_ref[...])

  return pl.pallas_call(
      body,
      grid=(n // TN,),
      in_specs=[
          pl.BlockSpec((TN, in_dim), lambda i: (i, 0)),
          pl.BlockSpec((TN, 3), lambda i: (i, 0)),
          pl.BlockSpec((in_dim, 64), lambda i: (0, 0)),
          pl.BlockSpec((1, 64), lambda i: (0, 0)),
      ],
      out_specs=[
          pl.BlockSpec((TN, 64), lambda i: (i, 0)),
          pl.BlockSpec((TN, TBW), lambda i: (i, 0)),
      ],
      out_shape=[
          jax.ShapeDtypeStruct((n, 64), jnp.float32),
          jax.ShapeDtypeStruct((n, TBW), jnp.bfloat16),
      ],
  )(x, pos, w, b)


def _tc_edge_mlp(gd, gs, ea, wts):
  e = gd.shape[0]
  w1a, w1b, w1e, w1d, b1, w2, b2, q1, q1b, q2, q2b = wts

  def body(gd_ref, gs_ref, ea_ref, w1a_ref, w1b_ref, w1e_ref, w1d_ref,
           b1_ref, w2_ref, b2_ref, q1_ref, q1b_ref, q2_ref, q2b_ref, o_ref):
    gd = gd_ref[...].astype(jnp.float32)
    gs = gs_ref[...].astype(jnp.float32)
    hd = gd[:, :64]
    hs = gs[:, :64]
    pd = gd[:, 64:67] + gd[:, 67:70]
    ps = gs[:, 64:67] + gs[:, 67:70]
    d = pd - ps
    dist2 = jnp.sum(d * d, axis=1, keepdims=True)
    x1 = (
        jnp.dot(hd, w1a_ref[...], preferred_element_type=jnp.float32)
        + jnp.dot(hs, w1b_ref[...], preferred_element_type=jnp.float32)
        + jnp.dot(ea_ref[...], w1e_ref[...], preferred_element_type=jnp.float32)
        + dist2 * w1d_ref[...]
        + b1_ref[...]
    )
    m = jnp.maximum(x1, 0.0)
    m = jnp.maximum(
        jnp.dot(m, w2_ref[...], preferred_element_type=jnp.float32)
        + b2_ref[...], 0.0)
    t = jnp.maximum(
        jnp.dot(m, q1_ref[...], preferred_element_type=jnp.float32)
        + q1b_ref[...], 0.0)
    w = jnp.sum(t * q2_ref[...], axis=1, keepdims=True) + q2b_ref[...]
    pmsg = d * w
    o_ref[0] = jnp.concatenate(
        [m[:, :32], pmsg, jnp.ones((TE, 1), jnp.float32),
         jnp.zeros((TE, PW - 36), jnp.float32)], axis=1)
    o_ref[1] = jnp.concatenate(
        [m[:, 32:], jnp.zeros((TE, PW - 32), jnp.float32)], axis=1)

  full = lambda shape: pl.BlockSpec(shape, lambda i: tuple(0 for _ in shape))
  return pl.pallas_call(
      body,
      grid=(e // TE,),
      in_specs=[
          pl.BlockSpec((TE, TBW), lambda i: (i, 0)),
          pl.BlockSpec((TE, TBW), lambda i: (i, 0)),
          pl.BlockSpec((TE, 4), lambda i: (i, 0)),
          full((64, 64)), full((64, 64)), full((4, 64)), full((1, 64)),
          full((1, 64)), full((64, 64)), full((1, 64)), full((64, 64)),
          full((1, 64)), full((1, 64)), full((1, 1)),
      ],
      out_specs=pl.BlockSpec((2, TE, PW), lambda i: (0, i, 0)),
      out_shape=jax.ShapeDtypeStruct((2, e, PW), jnp.float32),
  )(gd, gs, ea, w1a, w1b, w1e, w1d, b1, w2, b2, q1, q1b, q2, q2b)


def _tc_update(h, pos, acc, wts):
  n = h.shape[0]
  u1a, u1b, ub1, u2, ub2 = wts

  def body(h_ref, p_ref, a0_ref, a1_ref, u1a_ref, u1b_ref, ub1_ref, u2_ref,
           ub2_ref, ho_ref, po_ref, t_ref):
    h = h_ref[...]
    pos = p_ref[...]
    a0 = a0_ref[0]
    a1 = a1_ref[0]
    magg = jnp.concatenate([a0[:, :32], a1[:, :32]], axis=1)
    pos_sum = a0[:, 32:35]
    cnt = a0[:, 35:36]
    u = jnp.maximum(
        jnp.dot(h, u1a_ref[...], preferred_element_type=jnp.float32)
        + jnp.dot(magg, u1b_ref[...], preferred_element_type=jnp.float32)
        + ub1_ref[...], 0.0)
    h2 = h + jnp.dot(u, u2_ref[...], preferred_element_type=jnp.float32) \
        + ub2_ref[...]
    pos2 = pos + pos_sum / jnp.maximum(cnt, 1.0)
    ho_ref[...] = h2
    po_ref[...] = pos2
    t_ref[...] = _pack_table(h2, pos2)

  full = lambda shape: pl.BlockSpec(shape, lambda i: tuple(0 for _ in shape))
  return pl.pallas_call(
      body,
      grid=(n // TN,),
      in_specs=[
          pl.BlockSpec((TN, 64), lambda i: (i, 0)),
          pl.BlockSpec((TN, 3), lambda i: (i, 0)),
          pl.BlockSpec((1, TN, PW), lambda i: (0, i, 0)),
          pl.BlockSpec((1, TN, PW), lambda i: (1, i, 0)),
          full((64, 64)), full((64, 64)), full((1, 64)),
          full((64, 64)), full((1, 64)),
      ],
      out_specs=[
          pl.BlockSpec((TN, 64), lambda i: (i, 0)),
          pl.BlockSpec((TN, 3), lambda i: (i, 0)),
          pl.BlockSpec((TN, TBW), lambda i: (i, 0)),
      ],
      out_shape=[
          jax.ShapeDtypeStruct((n, 64), jnp.float32),
          jax.ShapeDtypeStruct((n, 3), jnp.float32),
          jax.ShapeDtypeStruct((n, TBW), jnp.bfloat16),
      ],
  )(h, pos, acc, acc, u1a, u1b, ub1, u2, ub2)


def _tc_pred(h, wp_row, bp):
  n = h.shape[0]

  def body(h_ref, w_ref, b_ref, o_ref):
    o_ref[...] = jnp.sum(
        h_ref[...] * w_ref[...], axis=1, keepdims=True) + b_ref[...]

  return pl.pallas_call(
      body,
      grid=(n // TN,),
      in_specs=[
          pl.BlockSpec((TN, 64), lambda i: (i, 0)),
          pl.BlockSpec((1, 64), lambda i: (0, 0)),
          pl.BlockSpec((1, 1), lambda i: (0, 0)),
      ],
      out_specs=pl.BlockSpec((TN, 1), lambda i: (i, 0)),
      out_shape=jax.ShapeDtypeStruct((n, 1), jnp.float32),
  )(h, wp_row, bp)


def kernel(x, pos, edge_index, edge_attr, params):
  n = x.shape[0]
  e = edge_attr.shape[0]
  src = edge_index[0]
  dst = edge_index[1]
  idx_d = dst.reshape(1, e)
  idx_s = src.reshape(1, e)
  idx2d = dst.reshape(e // CH, CH)
  zeros_n = jnp.zeros((n, PW), jnp.float32)

  w_in, b_in = params['lin_in']
  h, table = _tc_lin_in(x, pos, w_in, b_in.reshape(1, 64))

  for lp in params['layers']:
    w1, b1 = lp['msg1']
    w2, b2 = lp['msg2']
    q1, q1b = lp['pos1']
    q2, q2b = lp['pos2']
    u1, ub1 = lp['upd1']
    u2, ub2 = lp['upd2']
    gd, gs = _sc_gather2(table, idx_d, idx_s)
    payload = _tc_edge_mlp(
        gd, gs, edge_attr,
        (w1[:64], w1[64:128], w1[129:133], w1[128:129], b1.reshape(1, 64),
         w2, b2.reshape(1, 64), q1, q1b.reshape(1, 64),
         q2.reshape(1, 64), q2b.reshape(1, 1)))
    acc = _sc_scatter_add(payload, idx2d, zeros_n)
    h, pos, table = _tc_update(
        h, pos, acc,
        (u1[:64], u1[64:128], ub1.reshape(1, 64), u2, ub2.reshape(1, 64)))

  wp, bp = params['lin_pred']
  return _tc_pred(h, wp.reshape(1, 64), bp.reshape(1, 1))


# speedup vs baseline: 1.2202x; 1.2202x over previous
import functools

import jax
import jax.numpy as jnp
from jax import lax
from jax.experimental import pallas as pl
from jax.experimental.pallas import tpu as pltpu
from jax.experimental.pallas import tpu_sc as plsc

TBW = 80
PW = 40
GW = 128
CH = 64
TE = 2000
TN = 2000


def _sc_mesh():
  return plsc.VectorSubcoreMesh(core_axis_name="c", subcore_axis_name="s")


_SC_PARAMS = pltpu.CompilerParams(use_tc_tiling_on_sc=False)


def _sc_gather2(table, idx_d, idx_s):
  e = idx_d.shape[1]
  out_t = jax.ShapeDtypeStruct((e, TBW), jnp.float32)

  @functools.partial(
      pl.kernel,
      out_type=(out_t, out_t),
      mesh=_sc_mesh(),
      compiler_params=_SC_PARAMS,
  )
  def k(t_hbm, id_hbm, is_hbm, od_hbm, os_hbm):
    def body(id_v, is_v, od_v, os_v):
      pltpu.sync_copy(t_hbm.at[id_v.at[0]], od_v)
      pltpu.sync_copy(t_hbm.at[is_v.at[0]], os_v)

    pltpu.emit_pipeline(
        body,
        grid=(e // GW,),
        in_specs=[
            pl.BlockSpec((1, GW), lambda i: (0, i)),
            pl.BlockSpec((1, GW), lambda i: (0, i)),
        ],
        out_specs=[
            pl.BlockSpec((GW, TBW), lambda i: (i, 0)),
            pl.BlockSpec((GW, TBW), lambda i: (i, 0)),
        ],
        core_axis_name=("c", "s"),
        dimension_semantics=(pltpu.PARALLEL,),
    )(id_hbm, is_hbm, od_hbm, os_hbm)

  return k(table, idx_d, idx_s)


def _sc_scatter_add(payload, idx2d, zeros_n):
  e = payload.shape[1]
  n = zeros_n.shape[0]
  total_ch = e // CH
  rows_per = n // 16

  @functools.partial(
      pl.kernel,
      out_type=jax.ShapeDtypeStruct((2, n, PW), jnp.float32),
      mesh=_sc_mesh(),
      compiler_params=_SC_PARAMS,
      scratch_types=[
          pltpu.VMEM_SHARED((n, PW), jnp.float32),
          pltpu.VMEM((CH, PW), jnp.float32),
          pltpu.VMEM((CH, PW), jnp.float32),
          pltpu.VMEM((1, CH), jnp.int32),
          pltpu.VMEM((1, CH), jnp.int32),
          pltpu.SemaphoreType.DMA,
          pltpu.SemaphoreType.DMA,
          pltpu.SemaphoreType.DMA,
          pltpu.SemaphoreType.DMA,
          pltpu.SemaphoreType.DMA,
          pltpu.SemaphoreType.DMA,
      ],
  )
  def k(p_hbm, i_hbm, z_hbm, o_hbm, acc_sh, pay0, pay1, idx0, idx1,
        sp0, sp1, si0, si1, ss0, ss1):
    pay_b = (pay0, pay1)
    idx_b = (idx0, idx1)
    sp_b = (sp0, sp1)
    si_b = (si0, si1)
    ss_b = (ss0, ss1)
    c = lax.axis_index("c")
    s = lax.axis_index("s")
    pltpu.sync_copy(
        z_hbm.at[pl.ds(s * rows_per, rows_per)],
        acc_sh.at[pl.ds(s * rows_per, rows_per)],
    )
    plsc.subcore_barrier()

    n_trips = (total_ch - s + 15) // 16

    def start_in(t, b):
      ch = s + 16 * t
      pltpu.async_copy(i_hbm.at[pl.ds(ch, 1)], idx_b[b], si_b[b])
      pltpu.async_copy(
          p_hbm.at[c].at[pl.ds(ch * CH, CH)], pay_b[b], sp_b[b])

    def wait_in(t, b):
      ch = s + 16 * t
      pltpu.make_async_copy(
          i_hbm.at[pl.ds(ch, 1)], idx_b[b], si_b[b]).wait()
      pltpu.make_async_copy(
          p_hbm.at[c].at[pl.ds(ch * CH, CH)], pay_b[b], sp_b[b]).wait()

    def do_scatter(b):
      pltpu.async_copy(
          pay_b[b], acc_sh.at[idx_b[b].at[0]], ss_b[b], add=True).wait()

    for b in (0, 1):
      start_in(b, b)

    def pair(tp, carry):
      for b in (0, 1):
        t = 2 * tp + b
        wait_in(t, b)
        do_scatter(b)
        @pl.when(t + 2 < n_trips)
        def _():
          start_in(t + 2, b)
      return carry

    lax.fori_loop(0, n_trips // 2, pair, 0)

    @pl.when(n_trips % 2 == 1)
    def _():
      t = n_trips - 1
      wait_in(t, 0)
      do_scatter(0)

    plsc.subcore_barrier()
    pltpu.sync_copy(
        acc_sh.at[pl.ds(s * rows_per, rows_per)],
        o_hbm.at[c].at[pl.ds(s * rows_per, rows_per)],
    )

  return k(payload, idx2d, zeros_n)


def _pack_table(h, pos):
  b = h.shape[0]
  return jnp.concatenate(
      [h, pos, jnp.zeros((b, TBW - 67), jnp.float32)], axis=1)


def _tc_lin_in(x, pos, w, b):
  n, in_dim = x.shape

  def body(x_ref, p_ref, w_ref, b_ref, h_ref, t_ref):
    h = jnp.dot(x_ref[...], w_ref[...], preferred_element_type=jnp.float32)
    h = h + b_ref[...]
    h_ref[...] = h
    t_ref[...] = _pack_table(h, p_ref[...])

  return pl.pallas_call(
      body,
      grid=(n // TN,),
      in_specs=[
          pl.BlockSpec((TN, in_dim), lambda i: (i, 0)),
          pl.BlockSpec((TN, 3), lambda i: (i, 0)),
          pl.BlockSpec((in_dim, 64), lambda i: (0, 0)),
          pl.BlockSpec((1, 64), lambda i: (0, 0)),
      ],
      out_specs=[
          pl.BlockSpec((TN, 64), lambda i: (i, 0)),
          pl.BlockSpec((TN, TBW), lambda i: (i, 0)),
      ],
      out_shape=[
          jax.ShapeDtypeStruct((n, 64), jnp.float32),
          jax.ShapeDtypeStruct((n, TBW), jnp.float32),
      ],
  )(x, pos, w, b)


def _tc_edge_mlp(gd, gs, ea, wts):
  e = gd.shape[0]
  w1a, w1b, w1e, w1d, b1, w2, b2, q1, q1b, q2, q2b = wts

  def body(gd_ref, gs_ref, ea_ref, w1a_ref, w1b_ref, w1e_ref, w1d_ref,
           b1_ref, w2_ref, b2_ref, q1_ref, q1b_ref, q2_ref, q2b_ref, o_ref):
    gd = gd_ref[...]
    gs = gs_ref[...]
    hd = gd[:, :64]
    hs = gs[:, :64]
    pd = gd[:, 64:67]
    ps = gs[:, 64:67]
    d = pd - ps
    dist2 = jnp.sum(d * d, axis=1, keepdims=True)
    x1 = (
        jnp.dot(hd, w1a_ref[...], preferred_element_type=jnp.float32)
        + jnp.dot(hs, w1b_ref[...], preferred_element_type=jnp.float32)
        + jnp.dot(ea_ref[...], w1e_ref[...], preferred_element_type=jnp.float32)
        + dist2 * w1d_ref[...]
        + b1_ref[...]
    )
    m = jnp.maximum(x1, 0.0)
    m = jnp.maximum(
        jnp.dot(m, w2_ref[...], preferred_element_type=jnp.float32)
        + b2_ref[...], 0.0)
    t = jnp.maximum(
        jnp.dot(m, q1_ref[...], preferred_element_type=jnp.float32)
        + q1b_ref[...], 0.0)
    w = jnp.sum(t * q2_ref[...], axis=1, keepdims=True) + q2b_ref[...]
    pmsg = d * w
    o_ref[0] = jnp.concatenate(
        [m[:, :32], pmsg, jnp.ones((TE, 1), jnp.float32),
         jnp.zeros((TE, PW - 36), jnp.float32)], axis=1)
    o_ref[1] = jnp.concatenate(
        [m[:, 32:], jnp.zeros((TE, PW - 32), jnp.float32)], axis=1)

  full = lambda shape: pl.BlockSpec(shape, lambda i: tuple(0 for _ in shape))
  return pl.pallas_call(
      body,
      grid=(e // TE,),
      in_specs=[
          pl.BlockSpec((TE, TBW), lambda i: (i, 0)),
          pl.BlockSpec((TE, TBW), lambda i: (i, 0)),
          pl.BlockSpec((TE, 4), lambda i: (i, 0)),
          full((64, 64)), full((64, 64)), full((4, 64)), full((1, 64)),
          full((1, 64)), full((64, 64)), full((1, 64)), full((64, 64)),
          full((1, 64)), full((1, 64)), full((1, 1)),
      ],
      out_specs=pl.BlockSpec((2, TE, PW), lambda i: (0, i, 0)),
      out_shape=jax.ShapeDtypeStruct((2, e, PW), jnp.float32),
  )(gd, gs, ea, w1a, w1b, w1e, w1d, b1, w2, b2, q1, q1b, q2, q2b)


def _tc_update(h, pos, acc, wts):
  n = h.shape[0]
  u1a, u1b, ub1, u2, ub2 = wts

  def body(h_ref, p_ref, a0_ref, a1_ref, u1a_ref, u1b_ref, ub1_ref, u2_ref,
           ub2_ref, ho_ref, po_ref, t_ref):
    h = h_ref[...]
    pos = p_ref[...]
    a0 = a0_ref[0]
    a1 = a1_ref[0]
    magg = jnp.concatenate([a0[:, :32], a1[:, :32]], axis=1)
    pos_sum = a0[:, 32:35]
    cnt = a0[:, 35:36]
    u = jnp.maximum(
        jnp.dot(h, u1a_ref[...], preferred_element_type=jnp.float32)
        + jnp.dot(magg, u1b_ref[...], preferred_element_type=jnp.float32)
        + ub1_ref[...], 0.0)
    h2 = h + jnp.dot(u, u2_ref[...], preferred_element_type=jnp.float32) \
        + ub2_ref[...]
    pos2 = pos + pos_sum / jnp.maximum(cnt, 1.0)
    ho_ref[...] = h2
    po_ref[...] = pos2
    t_ref[...] = _pack_table(h2, pos2)

  full = lambda shape: pl.BlockSpec(shape, lambda i: tuple(0 for _ in shape))
  return pl.pallas_call(
      body,
      grid=(n // TN,),
      in_specs=[
          pl.BlockSpec((TN, 64), lambda i: (i, 0)),
          pl.BlockSpec((TN, 3), lambda i: (i, 0)),
          pl.BlockSpec((1, TN, PW), lambda i: (0, i, 0)),
          pl.BlockSpec((1, TN, PW), lambda i: (1, i, 0)),
          full((64, 64)), full((64, 64)), full((1, 64)),
          full((64, 64)), full((1, 64)),
      ],
      out_specs=[
          pl.BlockSpec((TN, 64), lambda i: (i, 0)),
          pl.BlockSpec((TN, 3), lambda i: (i, 0)),
          pl.BlockSpec((TN, TBW), lambda i: (i, 0)),
      ],
      out_shape=[
          jax.ShapeDtypeStruct((n, 64), jnp.float32),
          jax.ShapeDtypeStruct((n, 3), jnp.float32),
          jax.ShapeDtypeStruct((n, TBW), jnp.float32),
      ],
  )(h, pos, acc, acc, u1a, u1b, ub1, u2, ub2)


def _tc_pred(h, wp_row, bp):
  n = h.shape[0]

  def body(h_ref, w_ref, b_ref, o_ref):
    o_ref[...] = jnp.sum(
        h_ref[...] * w_ref[...], axis=1, keepdims=True) + b_ref[...]

  return pl.pallas_call(
      body,
      grid=(n // TN,),
      in_specs=[
          pl.BlockSpec((TN, 64), lambda i: (i, 0)),
          pl.BlockSpec((1, 64), lambda i: (0, 0)),
          pl.BlockSpec((1, 1), lambda i: (0, 0)),
      ],
      out_specs=pl.BlockSpec((TN, 1), lambda i: (i, 0)),
      out_shape=jax.ShapeDtypeStruct((n, 1), jnp.float32),
  )(h, wp_row, bp)


def kernel(x, pos, edge_index, edge_attr, params):
  n = x.shape[0]
  e = edge_attr.shape[0]
  src = edge_index[0]
  dst = edge_index[1]
  idx_d = dst.reshape(1, e)
  idx_s = src.reshape(1, e)
  idx2d = dst.reshape(e // CH, CH)
  zeros_n = jnp.zeros((n, PW), jnp.float32)

  w_in, b_in = params['lin_in']
  h, table = _tc_lin_in(x, pos, w_in, b_in.reshape(1, 64))

  for lp in params['layers']:
    w1, b1 = lp['msg1']
    w2, b2 = lp['msg2']
    q1, q1b = lp['pos1']
    q2, q2b = lp['pos2']
    u1, ub1 = lp['upd1']
    u2, ub2 = lp['upd2']
    gd, gs = _sc_gather2(table, idx_d, idx_s)
    payload = _tc_edge_mlp(
        gd, gs, edge_attr,
        (w1[:64], w1[64:128], w1[129:133], w1[128:129], b1.reshape(1, 64),
         w2, b2.reshape(1, 64), q1, q1b.reshape(1, 64),
         q2.reshape(1, 64), q2b.reshape(1, 1)))
    acc = _sc_scatter_add(payload, idx2d, zeros_n)
    h, pos, table = _tc_update(
        h, pos, acc,
        (u1[:64], u1[64:128], ub1.reshape(1, 64), u2, ub2.reshape(1, 64)))

  wp, bp = params['lin_pred']
  return _tc_pred(h, wp.reshape(1, 64), bp.reshape(1, 1))
